# Initial kernel scaffold; baseline (speedup 1.0000x reference)
import jax
import jax.numpy as jnp
from jax.experimental import pallas as pl


def kernel(node_attrs, node_feats, edge_attrs, edge_feats, edge_index, W_radial, W_linear, W_skip):
    raise NotImplementedError("write your pallas kernel here")


# speedup vs baseline: 2.2478x; 2.2478x over previous
import functools
import math

import jax
import jax.numpy as jnp
from jax import lax
from jax.experimental import pallas as pl
from jax.experimental.pallas import tpu as pltpu
from jax.experimental.pallas import tpu_sc as plsc

N_NODES = 10000
N_EDGES = 320000
D = 128
D_ATTR = 16
D_EF = 16

NC = 2
NS = 16
NW = NC * NS
E_PER_W = N_EDGES // NW
K = 80
NCHUNK = E_PER_W // K
N_PAD = 10240
ROWS_PER_TILE = N_PAD // NS
ZROWS = 128


EBLK = 2000


def _coef_body(ef_ref, ea_ref, wr_ref, c_ref):
    c = jnp.dot(ef_ref[...], wr_ref[...], preferred_element_type=jnp.float32)
    c_ref[...] = c * ea_ref[...] * (1.0 / math.sqrt(D_EF))


def _edge_coefs(edge_feats, edge_attrs, W_radial):
    return pl.pallas_call(
        _coef_body,
        grid=(N_EDGES // EBLK,),
        in_specs=[
            pl.BlockSpec((EBLK, D_EF), lambda i: (i, 0)),
            pl.BlockSpec((EBLK, 1), lambda i: (i, 0)),
            pl.BlockSpec((D_EF, D), lambda i: (0, 0)),
        ],
        out_specs=pl.BlockSpec((EBLK, D), lambda i: (i, 0)),
        out_shape=jax.ShapeDtypeStruct((N_EDGES, D), jnp.float32),
    )(edge_feats, edge_attrs, W_radial)


def _sc_body(nf_hbm, c_hbm, snd_hbm, rcv_hbm, out_hbm,
             snd_v, rcv_v, rows_v, c_v, zbuf_v, msg_sh, sem):
    cid = lax.axis_index("c")
    sid = lax.axis_index("s")
    wid = sid * NC + cid

    def _zrow(i, carry):
        for d8 in range(D // 16):
            zbuf_v[i, pl.ds(d8 * 16, 16)] = jnp.zeros((16,), jnp.float32)
        return carry

    lax.fori_loop(0, ZROWS, _zrow, 0)
    for j in range(ROWS_PER_TILE // ZROWS):
        pltpu.sync_copy(zbuf_v, msg_sh.at[pl.ds(sid * ROWS_PER_TILE + j * ZROWS, ZROWS)])
    plsc.subcore_barrier()

    ebase = wid * E_PER_W

    def _chunk(j, carry):
        b = ebase + j * K
        pltpu.sync_copy(snd_hbm.at[pl.ds(b, K)], snd_v)
        pltpu.sync_copy(rcv_hbm.at[pl.ds(b, K)], rcv_v)
        pltpu.async_copy(nf_hbm.at[snd_v], rows_v, sem).wait()
        pltpu.sync_copy(c_hbm.at[pl.ds(b, K)], c_v)

        def _mulrow(e, c2):
            for d8 in range(D // 16):
                sl = pl.ds(d8 * 16, 16)
                rows_v[e, sl] = rows_v[e, sl] * c_v[e, sl]
            return c2

        lax.fori_loop(0, K, _mulrow, 0)
        pltpu.sync_copy(rows_v, msg_sh.at[rcv_v], add=True)
        return carry

    lax.fori_loop(0, NCHUNK, _chunk, 0)
    plsc.subcore_barrier()

    for j in range(ROWS_PER_TILE // ZROWS):
        r0 = sid * ROWS_PER_TILE + j * ZROWS
        pltpu.sync_copy(msg_sh.at[pl.ds(r0, ZROWS)], out_hbm.at[cid, pl.ds(r0, ZROWS)])


def _sc_message(node_feats, c, snd, rcv):
    mesh = plsc.VectorSubcoreMesh(core_axis_name="c", subcore_axis_name="s")
    f = functools.partial(
        pl.kernel,
        out_type=jax.ShapeDtypeStruct((NC, N_PAD, D), jnp.float32),
        mesh=mesh,
        scratch_types=[
            pltpu.VMEM((K,), jnp.int32),
            pltpu.VMEM((K,), jnp.int32),
            pltpu.VMEM((K, D), jnp.float32),
            pltpu.VMEM((K, D), jnp.float32),
            pltpu.VMEM((ZROWS, D), jnp.float32),
            pltpu.MemorySpace.VMEM_SHARED((N_PAD, D), jnp.float32),
            pltpu.SemaphoreType.DMA,
        ],
    )(_sc_body)
    return f(node_feats, c, snd, rcv)


NBLK = 2000


def _final_body(p_ref, attrs_ref, wl_ref, wst_ref, out_ref):
    msg = p_ref[0] + p_ref[1]
    m2 = jnp.dot(msg, wl_ref[...], preferred_element_type=jnp.float32) * (
        1.0 / math.sqrt(D))
    acc = jnp.zeros((NBLK, D), jnp.float32)
    for v in range(D_ATTR):
        acc = acc + attrs_ref[:, v:v + 1] * jnp.dot(
            m2, wst_ref[v], preferred_element_type=jnp.float32)
    out_ref[...] = acc * (1.0 / math.sqrt(D * D_ATTR))


def _final(partials, node_attrs, W_linear, W_skip_t):
    return pl.pallas_call(
        _final_body,
        grid=(N_NODES // NBLK,),
        in_specs=[
            pl.BlockSpec((NC, NBLK, D), lambda i: (0, i, 0)),
            pl.BlockSpec((NBLK, D_ATTR), lambda i: (i, 0)),
            pl.BlockSpec((D, D), lambda i: (0, 0)),
            pl.BlockSpec((D_ATTR, D, D), lambda i: (0, 0, 0)),
        ],
        out_specs=pl.BlockSpec((NBLK, D), lambda i: (i, 0)),
        out_shape=jax.ShapeDtypeStruct((N_NODES, D), jnp.float32),
    )(partials, node_attrs, W_linear, W_skip_t)


def kernel(node_attrs, node_feats, edge_attrs, edge_feats, edge_index, W_radial, W_linear, W_skip):
    snd = edge_index[0]
    rcv = edge_index[1]
    c = _edge_coefs(edge_feats, edge_attrs, W_radial)
    partials = _sc_message(node_feats, c, snd, rcv)
    return _final(partials, node_attrs, W_linear, W_skip.transpose(1, 0, 2))


# speedup vs baseline: 3.3035x; 1.4696x over previous
import functools
import math

import jax
import jax.numpy as jnp
from jax import lax
from jax.experimental import pallas as pl
from jax.experimental.pallas import tpu as pltpu
from jax.experimental.pallas import tpu_sc as plsc

N_NODES = 10000
N_EDGES = 320000
D = 128
D_ATTR = 16
D_EF = 16

NC = 2
NS = 16
NW = NC * NS
E_PER_W = N_EDGES // NW
K = 80
NCHUNK = E_PER_W // K
N_PAD = 10240
ROWS_PER_TILE = N_PAD // NS
ZROWS = 32


EBLK = 2000


def _coef_body(ef_ref, ea_ref, wr_ref, c_ref):
    c = jnp.dot(ef_ref[...], wr_ref[...], preferred_element_type=jnp.float32)
    c_ref[...] = c * ea_ref[...] * (1.0 / math.sqrt(D_EF))


def _edge_coefs(edge_feats, edge_attrs, W_radial):
    return pl.pallas_call(
        _coef_body,
        grid=(N_EDGES // EBLK,),
        in_specs=[
            pl.BlockSpec((EBLK, D_EF), lambda i: (i, 0)),
            pl.BlockSpec((EBLK, 1), lambda i: (i, 0)),
            pl.BlockSpec((D_EF, D), lambda i: (0, 0)),
        ],
        out_specs=pl.BlockSpec((EBLK, D), lambda i: (i, 0)),
        out_shape=jax.ShapeDtypeStruct((N_EDGES, D), jnp.float32),
    )(edge_feats, edge_attrs, W_radial)


def _sc_body(nf_hbm, c_hbm, snd_hbm, rcv_hbm, out_hbm,
             snd_v, rcv_v, rows_v, c_v, zbuf_v, msg_sh,
             isem0, isem1, gsem0, gsem1, csem0, csem1):
    cid = lax.axis_index("c")
    sid = lax.axis_index("s")
    wid = sid * NC + cid
    isems = (isem0, isem1)
    gsems = (gsem0, gsem1)
    csems = (csem0, csem1)
    ebase = wid * E_PER_W

    def _start_idx(j, p):
        b = ebase + j * K
        pltpu.async_copy(snd_hbm.at[pl.ds(b, K)], snd_v.at[p], isems[p])
        pltpu.async_copy(rcv_hbm.at[pl.ds(b, K)], rcv_v.at[p], isems[p])

    def _wait_idx(j, p):
        b = ebase + j * K
        pltpu.make_async_copy(snd_hbm.at[pl.ds(b, K)], snd_v.at[p], isems[p]).wait()
        pltpu.make_async_copy(rcv_hbm.at[pl.ds(b, K)], rcv_v.at[p], isems[p]).wait()

    def _start_data(j, p):
        pltpu.async_copy(nf_hbm.at[snd_v.at[p]], rows_v.at[p], gsems[p])
        pltpu.async_copy(c_hbm.at[pl.ds(ebase + j * K, K)], c_v.at[p], csems[p])

    def _wait_data(j, p):
        pltpu.make_async_copy(nf_hbm.at[snd_v.at[p]], rows_v.at[p], gsems[p]).wait()
        pltpu.make_async_copy(c_hbm.at[pl.ds(ebase + j * K, K)], c_v.at[p], csems[p]).wait()

    _start_idx(0, 0)
    _wait_idx(0, 0)
    _start_data(0, 0)
    _start_idx(1, 1)

    def _zrow(i, carry):
        for d8 in range(D // 16):
            zbuf_v[i, pl.ds(d8 * 16, 16)] = jnp.zeros((16,), jnp.float32)
        return carry

    lax.fori_loop(0, ZROWS, _zrow, 0)
    for j in range(ROWS_PER_TILE // ZROWS):
        pltpu.sync_copy(zbuf_v, msg_sh.at[pl.ds(sid * ROWS_PER_TILE + j * ZROWS, ZROWS)])
    plsc.subcore_barrier()

    def _step(jj, p):
        @pl.when(jj < NCHUNK)
        def _():
            q = 1 - p

            @pl.when(jj + 1 < NCHUNK)
            def _():
                _wait_idx(jj + 1, q)
                _start_data(jj + 1, q)

            _wait_data(jj, p)

            def _mulrow(e, c2):
                for d8 in range(D // 16):
                    sl = pl.ds(d8 * 16, 16)
                    rows_v[p, e, sl] = rows_v[p, e, sl] * c_v[p, e, sl]
                return c2

            lax.fori_loop(0, K, _mulrow, 0)
            pltpu.sync_copy(rows_v.at[p], msg_sh.at[rcv_v.at[p]], add=True)

            @pl.when(jj + 2 < NCHUNK)
            def _():
                _start_idx(jj + 2, p)

    def _pair(j2, carry):
        _step(2 * j2, 0)
        _step(2 * j2 + 1, 1)
        return carry

    lax.fori_loop(0, (NCHUNK + 1) // 2, _pair, 0)
    plsc.subcore_barrier()

    for j in range(ROWS_PER_TILE // ZROWS):
        r0 = sid * ROWS_PER_TILE + j * ZROWS
        pltpu.sync_copy(msg_sh.at[pl.ds(r0, ZROWS)], out_hbm.at[cid, pl.ds(r0, ZROWS)])


def _sc_message(node_feats, c, snd, rcv):
    mesh = plsc.VectorSubcoreMesh(core_axis_name="c", subcore_axis_name="s")
    f = functools.partial(
        pl.kernel,
        out_type=jax.ShapeDtypeStruct((NC, N_PAD, D), jnp.float32),
        mesh=mesh,
        scratch_types=[
            pltpu.VMEM((2, K), jnp.int32),
            pltpu.VMEM((2, K), jnp.int32),
            pltpu.VMEM((2, K, D), jnp.float32),
            pltpu.VMEM((2, K, D), jnp.float32),
            pltpu.VMEM((ZROWS, D), jnp.float32),
            pltpu.MemorySpace.VMEM_SHARED((N_PAD, D), jnp.float32),
            pltpu.SemaphoreType.DMA,
            pltpu.SemaphoreType.DMA,
            pltpu.SemaphoreType.DMA,
            pltpu.SemaphoreType.DMA,
            pltpu.SemaphoreType.DMA,
            pltpu.SemaphoreType.DMA,
        ],
    )(_sc_body)
    return f(node_feats, c, snd, rcv)


NBLK = 2000


def _final_body(p_ref, attrs_ref, wl_ref, wst_ref, out_ref):
    msg = p_ref[0] + p_ref[1]
    m2 = jnp.dot(msg, wl_ref[...], preferred_element_type=jnp.float32) * (
        1.0 / math.sqrt(D))
    acc = jnp.zeros((NBLK, D), jnp.float32)
    for v in range(D_ATTR):
        acc = acc + attrs_ref[:, v:v + 1] * jnp.dot(
            m2, wst_ref[v], preferred_element_type=jnp.float32)
    out_ref[...] = acc * (1.0 / math.sqrt(D * D_ATTR))


def _final(partials, node_attrs, W_linear, W_skip_t):
    return pl.pallas_call(
        _final_body,
        grid=(N_NODES // NBLK,),
        in_specs=[
            pl.BlockSpec((NC, NBLK, D), lambda i: (0, i, 0)),
            pl.BlockSpec((NBLK, D_ATTR), lambda i: (i, 0)),
            pl.BlockSpec((D, D), lambda i: (0, 0)),
            pl.BlockSpec((D_ATTR, D, D), lambda i: (0, 0, 0)),
        ],
        out_specs=pl.BlockSpec((NBLK, D), lambda i: (i, 0)),
        out_shape=jax.ShapeDtypeStruct((N_NODES, D), jnp.float32),
    )(partials, node_attrs, W_linear, W_skip_t)


def kernel(node_attrs, node_feats, edge_attrs, edge_feats, edge_index, W_radial, W_linear, W_skip):
    snd = edge_index[0]
    rcv = edge_index[1]
    c = _edge_coefs(edge_feats, edge_attrs, W_radial)
    partials = _sc_message(node_feats, c, snd, rcv)
    return _final(partials, node_attrs, W_linear, W_skip.transpose(1, 0, 2))
